# Initial kernel scaffold; baseline (speedup 1.0000x reference)
import jax
import jax.numpy as jnp
from jax.experimental import pallas as pl


def kernel(x, target):
    raise NotImplementedError("write your pallas kernel here")


# speedup vs baseline: 6.9936x; 6.9936x over previous
import functools

import jax
import jax.numpy as jnp
from jax.experimental import pallas as pl

_VOCAB = 32000
_PAD = 0
_SMOOTH = 0.1
_CONF = 1.0 - _SMOOTH
_N = 2048
_SMOOTH_VAL = _SMOOTH / (_VOCAB - 2)

_ROW_BLOCK = 64


def _loss_block(tgt_ref, x_ref, out_ref):
    i = pl.program_id(0)
    x = x_ref[...]
    tgt = tgt_ref[...]
    cols = jax.lax.broadcasted_iota(jnp.int32, x.shape, 1)
    valid = tgt != _PAD
    is_t = cols == tgt
    w = jnp.where(is_t, -_CONF, -_SMOOTH_VAL)
    w = jnp.where(cols == _PAD, 0.0, w)
    w = jnp.where(valid, w, 0.0)
    const = jnp.float32(
        (_VOCAB - 2) * _SMOOTH_VAL * jnp.log(_SMOOTH_VAL)
        + _CONF * jnp.log(_CONF)
    )
    partial = jnp.sum(w * x) + const * jnp.sum(valid.astype(jnp.float32))
    partial = jnp.reshape(partial, (1, 1))

    @pl.when(i == 0)
    def _():
        out_ref[...] = jnp.zeros((1, 1), jnp.float32)

    out_ref[...] += partial


@functools.partial(jax.jit, static_argnames=())
def kernel(x, target):
    n, v = x.shape
    tgt2d = target.astype(jnp.int32).reshape(n, 1)
    grid = (n // _ROW_BLOCK,)
    out = pl.pallas_call(
        _loss_block,
        grid=grid,
        in_specs=[
            pl.BlockSpec((_ROW_BLOCK, 1), lambda i: (i, 0)),
            pl.BlockSpec((_ROW_BLOCK, v), lambda i: (i, 0)),
        ],
        out_specs=pl.BlockSpec((1, 1), lambda i: (0, 0)),
        out_shape=jax.ShapeDtypeStruct((1, 1), jnp.float32),
    )(tgt2d, x)
    return out[0, 0]


# speedup vs baseline: 7.8350x; 1.1203x over previous
import functools

import jax
import jax.numpy as jnp
from jax.experimental import pallas as pl

_VOCAB = 32000
_PAD = 0
_SMOOTH = 0.1
_CONF = 1.0 - _SMOOTH
_N = 2048
_SMOOTH_VAL = _SMOOTH / (_VOCAB - 2)

_ROW_BLOCK = 64


def _loss_block(tgt_ref, x_ref, out_ref):
    i = pl.program_id(0)
    x = x_ref[...]
    tgt = tgt_ref[...]
    cols = jax.lax.broadcasted_iota(jnp.int32, x.shape, 1)
    valid = tgt != _PAD
    s = jnp.sum(x, axis=1, keepdims=True)
    t = jnp.sum(jnp.where(cols == tgt, x, 0.0), axis=1, keepdims=True)
    x0 = x[:, 0:1]
    const = jnp.float32(
        (_VOCAB - 2) * _SMOOTH_VAL * jnp.log(_SMOOTH_VAL)
        + _CONF * jnp.log(_CONF)
    )
    contrib = const - _SMOOTH_VAL * (s - x0) + (_SMOOTH_VAL - _CONF) * t
    partial = jnp.sum(jnp.where(valid, contrib, 0.0))
    partial = jnp.reshape(partial, (1, 1))

    @pl.when(i == 0)
    def _():
        out_ref[...] = jnp.zeros((1, 1), jnp.float32)

    out_ref[...] += partial


@functools.partial(jax.jit, static_argnames=())
def kernel(x, target):
    n, v = x.shape
    tgt2d = target.astype(jnp.int32).reshape(n, 1)
    grid = (n // _ROW_BLOCK,)
    out = pl.pallas_call(
        _loss_block,
        grid=grid,
        in_specs=[
            pl.BlockSpec((_ROW_BLOCK, 1), lambda i: (i, 0)),
            pl.BlockSpec((_ROW_BLOCK, v), lambda i: (i, 0)),
        ],
        out_specs=pl.BlockSpec((1, 1), lambda i: (0, 0)),
        out_shape=jax.ShapeDtypeStruct((1, 1), jnp.float32),
    )(tgt2d, x)
    return out[0, 0]


# speedup vs baseline: 8.2497x; 1.0529x over previous
import functools

import jax
import jax.numpy as jnp
from jax.experimental import pallas as pl

_VOCAB = 32000
_PAD = 0
_SMOOTH = 0.1
_CONF = 1.0 - _SMOOTH
_N = 2048
_SMOOTH_VAL = _SMOOTH / (_VOCAB - 2)

_ROW_BLOCK = 128


def _loss_block(tgt_ref, x_ref, out_ref):
    i = pl.program_id(0)
    x = x_ref[...]
    tgt = tgt_ref[...]
    cols = jax.lax.broadcasted_iota(jnp.int32, x.shape, 1)
    valid = tgt != _PAD
    s = jnp.sum(x, axis=1, keepdims=True)
    t = jnp.sum(jnp.where(cols == tgt, x, 0.0), axis=1, keepdims=True)
    x0 = x[:, 0:1]
    const = jnp.float32(
        (_VOCAB - 2) * _SMOOTH_VAL * jnp.log(_SMOOTH_VAL)
        + _CONF * jnp.log(_CONF)
    )
    contrib = const - _SMOOTH_VAL * (s - x0) + (_SMOOTH_VAL - _CONF) * t
    partial = jnp.sum(jnp.where(valid, contrib, 0.0))
    partial = jnp.reshape(partial, (1, 1))

    @pl.when(i == 0)
    def _():
        out_ref[...] = jnp.zeros((1, 1), jnp.float32)

    out_ref[...] += partial


@functools.partial(jax.jit, static_argnames=())
def kernel(x, target):
    n, v = x.shape
    tgt2d = target.astype(jnp.int32).reshape(n, 1)
    grid = (n // _ROW_BLOCK,)
    out = pl.pallas_call(
        _loss_block,
        grid=grid,
        in_specs=[
            pl.BlockSpec((_ROW_BLOCK, 1), lambda i: (i, 0)),
            pl.BlockSpec((_ROW_BLOCK, v), lambda i: (i, 0)),
        ],
        out_specs=pl.BlockSpec((1, 1), lambda i: (0, 0)),
        out_shape=jax.ShapeDtypeStruct((1, 1), jnp.float32),
    )(tgt2d, x)
    return out[0, 0]
